# baseline (speedup 1.0000x reference)
import functools

import jax
import jax.numpy as jnp
from jax import lax
from jax.experimental import pallas as pl
from jax.experimental.pallas import tpu as pltpu
from jax.experimental.pallas import tpu_sc as plsc

_NBUF = 8
_CJ = 16


def _sc_add_rel_pos(x, table, *, s, s2, d, nrows, maxrel, nw, cj):
    rows_per_w = s // nw
    nchunks = s2 // cj
    nchunk_tot = rows_per_w * nchunks
    mesh = plsc.VectorSubcoreMesh(core_axis_name="c", subcore_axis_name="s")

    @functools.partial(
        pl.kernel,
        mesh=mesh,
        out_type=jax.ShapeDtypeStruct((s, s2, d), jnp.float32),
        scratch_types=(
            [pltpu.VMEM((nrows, d), jnp.float32)]
            + [pltpu.VMEM((cj, d), jnp.float32) for _ in range(_NBUF)]
            + [pltpu.SemaphoreType.DMA for _ in range(2 * _NBUF)]
        ),
    )
    def k(x_hbm, t_hbm, o_hbm, t_v, *bufs_and_sems):
        bufs = bufs_and_sems[:_NBUF]
        lsems = bufs_and_sems[_NBUF:2 * _NBUF]
        ssems = bufs_and_sems[2 * _NBUF:3 * _NBUF]
        wid = lax.axis_index("s") * 2 + lax.axis_index("c")
        pltpu.sync_copy(t_hbm, t_v)
        i0 = wid * rows_per_w

        def chunk_slice(cc):
            i = i0 + cc // nchunks
            j0 = (cc % nchunks) * cj
            return i, j0

        def start_load(cc, slot):
            i, j0 = chunk_slice(cc)
            pltpu.async_copy(
                x_hbm.at[i, pl.ds(j0, cj)], bufs[slot], lsems[slot]
            )

        def wait_load(slot):
            pltpu.make_async_copy(
                x_hbm.at[0, pl.ds(0, cj)], bufs[slot], lsems[slot]
            ).wait()

        def start_store(cc, slot):
            i, j0 = chunk_slice(cc)
            pltpu.async_copy(
                bufs[slot], o_hbm.at[i, pl.ds(j0, cj)], ssems[slot]
            )

        def wait_store(slot):
            pltpu.make_async_copy(
                bufs[slot], o_hbm.at[0, pl.ds(0, cj)], ssems[slot]
            ).wait()

        def compute(cc, slot):
            buf = bufs[slot]
            i, j0 = chunk_slice(cc)
            uni = jnp.logical_or(j0 >= i + maxrel, j0 + cj - 1 <= i - maxrel)

            def uni_fn():
                r = jnp.clip(i - j0, -maxrel, maxrel) + maxrel
                for db in range(d // 256):
                    off = db * 256
                    tv = [t_v[r, pl.ds(off + q * 16, 16)] for q in range(16)]

                    @plsc.parallel_loop(0, cj)
                    def _(j):
                        for q in range(16):
                            sl = pl.ds(off + q * 16, 16)
                            buf[j, sl] = buf[j, sl] + tv[q]

            def gen_fn():
                @plsc.parallel_loop(0, cj)
                def _(j):
                    r = jnp.clip(i - (j0 + j), -maxrel, maxrel) + maxrel
                    for dd in range(0, d, 16):
                        sl = pl.ds(dd, 16)
                        buf[j, sl] = buf[j, sl] + t_v[r, sl]

            lax.cond(uni, uni_fn, gen_fn)

        pf = _NBUF - 2
        for p in range(pf):
            start_load(p, p)

        @pl.loop(0, nchunk_tot, step=_NBUF)
        def _(cc0):
            for b in range(_NBUF):
                cc = cc0 + b
                pslot = (b + pf) % _NBUF

                @pl.when(cc + pf < nchunk_tot)
                def _():
                    @pl.when(cc >= 2)
                    def _():
                        wait_store(pslot)

                    start_load(cc + pf, pslot)

                wait_load(b)
                compute(cc, b)
                start_store(cc, b)

        for b in range(_NBUF):
            wait_store(b)

    return k(x, table)


@jax.jit
def kernel(x, table):
    s, s2, d = x.shape
    nrows = table.shape[0]
    maxrel = (nrows - 1) // 2
    return _sc_add_rel_pos(
        x, table, s=s, s2=s2, d=d, nrows=nrows, maxrel=maxrel, nw=32, cj=_CJ
    )


# speedup vs baseline: 1.0230x; 1.0230x over previous
import functools

import jax
import jax.numpy as jnp
from jax import lax
from jax.experimental import pallas as pl
from jax.experimental.pallas import tpu as pltpu
from jax.experimental.pallas import tpu_sc as plsc

_NBUF = 4
_CJ = 32


def _sc_add_rel_pos(x, table, *, s, s2, d, nrows, maxrel, nw, cj):
    rows_per_w = s // nw
    nchunks = s2 // cj
    nchunk_tot = rows_per_w * nchunks
    mesh = plsc.VectorSubcoreMesh(core_axis_name="c", subcore_axis_name="s")

    @functools.partial(
        pl.kernel,
        mesh=mesh,
        out_type=jax.ShapeDtypeStruct((s, s2, d), jnp.float32),
        scratch_types=(
            [pltpu.VMEM((nrows, d), jnp.float32)]
            + [pltpu.VMEM((cj, d), jnp.float32) for _ in range(_NBUF)]
            + [pltpu.SemaphoreType.DMA for _ in range(2 * _NBUF)]
        ),
    )
    def k(x_hbm, t_hbm, o_hbm, t_v, *bufs_and_sems):
        bufs = bufs_and_sems[:_NBUF]
        lsems = bufs_and_sems[_NBUF:2 * _NBUF]
        ssems = bufs_and_sems[2 * _NBUF:3 * _NBUF]
        wid = lax.axis_index("s") * 2 + lax.axis_index("c")
        pltpu.sync_copy(t_hbm, t_v)
        i0 = wid * rows_per_w

        def chunk_slice(cc):
            i = i0 + cc // nchunks
            j0 = (cc % nchunks) * cj
            return i, j0

        def start_load(cc, slot):
            i, j0 = chunk_slice(cc)
            pltpu.async_copy(
                x_hbm.at[i, pl.ds(j0, cj)], bufs[slot], lsems[slot]
            )

        def wait_load(slot):
            pltpu.make_async_copy(
                x_hbm.at[0, pl.ds(0, cj)], bufs[slot], lsems[slot]
            ).wait()

        def start_store(cc, slot):
            i, j0 = chunk_slice(cc)
            pltpu.async_copy(
                bufs[slot], o_hbm.at[i, pl.ds(j0, cj)], ssems[slot]
            )

        def wait_store(slot):
            pltpu.make_async_copy(
                bufs[slot], o_hbm.at[0, pl.ds(0, cj)], ssems[slot]
            ).wait()

        def compute(cc, slot):
            buf = bufs[slot]
            i, j0 = chunk_slice(cc)
            uni = jnp.logical_or(j0 >= i + maxrel, j0 + cj - 1 <= i - maxrel)

            def uni_fn():
                r = jnp.clip(i - j0, -maxrel, maxrel) + maxrel
                for db in range(d // 256):
                    off = db * 256
                    tv = [t_v[r, pl.ds(off + q * 16, 16)] for q in range(16)]

                    @plsc.parallel_loop(0, cj)
                    def _(j):
                        for q in range(16):
                            sl = pl.ds(off + q * 16, 16)
                            buf[j, sl] = buf[j, sl] + tv[q]

            def gen_fn():
                @plsc.parallel_loop(0, cj)
                def _(j):
                    r = jnp.clip(i - (j0 + j), -maxrel, maxrel) + maxrel
                    for dd in range(0, d, 16):
                        sl = pl.ds(dd, 16)
                        buf[j, sl] = buf[j, sl] + t_v[r, sl]

            lax.cond(uni, uni_fn, gen_fn)

        pf = _NBUF - 2
        for p in range(pf):
            start_load(p, p)

        @pl.loop(0, nchunk_tot, step=_NBUF)
        def _(cc0):
            for b in range(_NBUF):
                cc = cc0 + b
                pslot = (b + pf) % _NBUF

                @pl.when(cc + pf < nchunk_tot)
                def _():
                    @pl.when(cc >= 2)
                    def _():
                        wait_store(pslot)

                    start_load(cc + pf, pslot)

                wait_load(b)
                compute(cc, b)
                start_store(cc, b)

        for b in range(_NBUF):
            wait_store(b)

    return k(x, table)


@jax.jit
def kernel(x, table):
    s, s2, d = x.shape
    nrows = table.shape[0]
    maxrel = (nrows - 1) // 2
    return _sc_add_rel_pos(
        x, table, s=s, s2=s2, d=d, nrows=nrows, maxrel=maxrel, nw=32, cj=_CJ
    )


# speedup vs baseline: 1.0277x; 1.0046x over previous
import functools

import jax
import jax.numpy as jnp
from jax import lax
from jax.experimental import pallas as pl
from jax.experimental.pallas import tpu as pltpu
from jax.experimental.pallas import tpu_sc as plsc

_NBUF = 4
_CJ = 32


def _sc_add_rel_pos(x, table, *, s, s2, d, nrows, maxrel, nw, cj):
    rows_per_w = s // nw
    nchunks = s2 // cj
    nchunk_tot = rows_per_w * nchunks
    mesh = plsc.VectorSubcoreMesh(core_axis_name="c", subcore_axis_name="s")

    @functools.partial(
        pl.kernel,
        mesh=mesh,
        out_type=jax.ShapeDtypeStruct((s, s2, d), jnp.float32),
        scratch_types=(
            [pltpu.VMEM((nrows, d), jnp.float32)]
            + [pltpu.VMEM((cj, d), jnp.float32) for _ in range(_NBUF)]
            + [pltpu.SemaphoreType.DMA for _ in range(2 * _NBUF)]
        ),
    )
    def k(x_hbm, t_hbm, o_hbm, t_v, *bufs_and_sems):
        bufs = bufs_and_sems[:_NBUF]
        lsems = bufs_and_sems[_NBUF:2 * _NBUF]
        ssems = bufs_and_sems[2 * _NBUF:3 * _NBUF]
        wid = lax.axis_index("s") * 2 + lax.axis_index("c")
        pltpu.sync_copy(t_hbm, t_v)
        i0 = wid * rows_per_w

        def chunk_slice(cc):
            i = i0 + cc // nchunks
            j0 = (cc % nchunks) * cj
            return i, j0

        def start_load(cc, slot):
            i, j0 = chunk_slice(cc)
            pltpu.async_copy(
                x_hbm.at[i, pl.ds(j0, cj)], bufs[slot], lsems[slot]
            )

        def wait_load(slot):
            pltpu.make_async_copy(
                x_hbm.at[0, pl.ds(0, cj)], bufs[slot], lsems[slot]
            ).wait()

        def start_store(cc, slot):
            i, j0 = chunk_slice(cc)
            pltpu.async_copy(
                bufs[slot], o_hbm.at[i, pl.ds(j0, cj)], ssems[slot]
            )

        def wait_store(slot):
            pltpu.make_async_copy(
                bufs[slot], o_hbm.at[0, pl.ds(0, cj)], ssems[slot]
            ).wait()

        def compute(cc, slot):
            buf = bufs[slot]
            i, j0 = chunk_slice(cc)
            uni = jnp.logical_or(j0 >= i + maxrel, j0 + cj - 1 <= i - maxrel)

            def uni_fn():
                r = jnp.clip(i - j0, -maxrel, maxrel) + maxrel
                for db in range(d // 256):
                    off = db * 256
                    tv = [t_v[r, pl.ds(off + q * 16, 16)] for q in range(16)]

                    @plsc.parallel_loop(0, cj)
                    def _(j):
                        for q in range(16):
                            sl = pl.ds(off + q * 16, 16)
                            plsc.addupdate(buf.at[j, sl], tv[q])

            def gen_fn():
                @plsc.parallel_loop(0, cj)
                def _(j):
                    r = jnp.clip(i - (j0 + j), -maxrel, maxrel) + maxrel
                    for dd in range(0, d, 16):
                        sl = pl.ds(dd, 16)
                        plsc.addupdate(buf.at[j, sl], t_v[r, sl])

            lax.cond(uni, uni_fn, gen_fn)

        pf = _NBUF - 2
        for p in range(pf):
            start_load(p, p)

        @pl.loop(0, nchunk_tot, step=_NBUF)
        def _(cc0):
            for b in range(_NBUF):
                cc = cc0 + b
                pslot = (b + pf) % _NBUF

                @pl.when(cc + pf < nchunk_tot)
                def _():
                    @pl.when(cc >= 2)
                    def _():
                        wait_store(pslot)

                    start_load(cc + pf, pslot)

                wait_load(b)
                compute(cc, b)
                start_store(cc, b)

        for b in range(_NBUF):
            wait_store(b)

    return k(x, table)


@jax.jit
def kernel(x, table):
    s, s2, d = x.shape
    nrows = table.shape[0]
    maxrel = (nrows - 1) // 2
    return _sc_add_rel_pos(
        x, table, s=s, s2=s2, d=d, nrows=nrows, maxrel=maxrel, nw=32, cj=_CJ
    )
